# baseline (speedup 1.0000x reference)
import functools

import jax
import jax.numpy as jnp
from jax import lax
from jax.experimental import pallas as pl
from jax.experimental.pallas import tpu as pltpu
from jax.experimental.pallas import tpu_sc as plsc

_ROW = 768
_PROW = 1536
_CHP = 16
_NB = 4
_L = 16


def _build_sc_call(B, NC, NS, num_kp, d_dist, d_state):
    NW = NC * NS
    b_per_w = B // NW
    p_per_w = b_per_w // 2
    n_ch = p_per_w // _CHP
    d_out = d_dist + d_state
    mesh = plsc.VectorSubcoreMesh(core_axis_name="c", subcore_axis_name="s")

    @functools.partial(
        pl.kernel,
        mesh=mesh,
        out_type=(
            jax.ShapeDtypeStruct((B // 2, _PROW), jnp.float32),
            jax.ShapeDtypeStruct((NW * 4, _PROW), jnp.float32),
        ),
        scratch_types=[
            pltpu.VMEM((num_kp * d_dist,), jnp.float32),
            pltpu.VMEM((2 * d_state,), jnp.float32),
            pltpu.VMEM((4, _PROW), jnp.float32),
            pltpu.VMEM((p_per_w,), jnp.int32),
            pltpu.VMEM((n_ch, _CHP), jnp.int32),
            pltpu.VMEM((_NB, _CHP, _PROW), jnp.float32),
        ] + [pltpu.SemaphoreType.DMA] * (2 * _NB),
    )
    def sc_gather(dist_hbm, state_hbm, idx_hbm, out_hbm, table_hbm,
                  dist_v, state_v, fusedp_v, idx_v, idxp_v, rows_v, *sems):
        wid = lax.axis_index("s") * NC + lax.axis_index("c")
        base = wid * p_per_w

        pltpu.sync_copy(dist_hbm, dist_v)
        pltpu.sync_copy(state_hbm, state_v)
        pltpu.sync_copy(idx_hbm.at[pl.ds(wid * p_per_w, p_per_w)], idx_v)

        for m in range(4):
            for h in range(2):
                g = (m >> h) & 1
                half = h * _ROW
                for k in range(num_kp):
                    col = half + k * d_out
                    for j in range(d_dist // _L):
                        fusedp_v[m, pl.ds(col + j * _L, _L)] = (
                            dist_v[pl.ds(k * d_dist + j * _L, _L)])
                    for j in range(d_state // _L):
                        fusedp_v[m, pl.ds(col + d_dist + j * _L, _L)] = (
                            state_v[pl.ds(g * d_state + j * _L, _L)])
        pltpu.sync_copy(fusedp_v, table_hbm.at[pl.ds(4 * wid, 4)])

        off4 = jnp.broadcast_to(4 * wid, (_L,)).astype(jnp.int32)
        for c in range(n_ch):
            idxp_v[c, :] = idx_v[pl.ds(c * _CHP, _L)] + off4

        gsem = sems[:_NB]
        ssem = sems[_NB:]
        gat = [None] * _NB
        sca = [None] * _NB
        for c in range(min(_NB, n_ch)):
            gat[c] = pltpu.async_copy(
                table_hbm.at[idxp_v.at[c]], rows_v.at[c], gsem[c])
        for c in range(n_ch):
            p = c % _NB
            gat[p].wait()
            sca[p] = pltpu.async_copy(
                rows_v.at[p], out_hbm.at[pl.ds(base + c * _CHP, _CHP)],
                ssem[p])
            if c + _NB < n_ch:
                sca[p].wait()
                gat[p] = pltpu.async_copy(
                    table_hbm.at[idxp_v.at[c + _NB]], rows_v.at[p], gsem[p])
                sca[p] = None
        for p in range(_NB):
            if sca[p] is not None:
                sca[p].wait()

    return sc_gather


def kernel(grip_state, distinction_table, state_table):
    B = grip_state.shape[0]
    num_kp, d_dist = distinction_table.shape
    d_state = state_table.shape[-1]
    info = plsc.get_sparse_core_info()
    NC, NS = info.num_cores, info.num_subcores

    g = grip_state.astype(jnp.int32)
    pair_combo = g[0::2] + 2 * g[1::2]
    out, _ = _build_sc_call(B, NC, NS, num_kp, d_dist, d_state)(
        distinction_table.reshape(-1),
        state_table.reshape(-1),
        pair_combo)
    return out.reshape(B, num_kp, d_dist + d_state)
